# Initial kernel scaffold; baseline (speedup 1.0000x reference)
import jax
import jax.numpy as jnp
from jax.experimental import pallas as pl


def kernel(voxel_features, coors, batch_size, W_subm0, W_conv1, W_subm1, W_subm2, W_conv2):
    raise NotImplementedError("write your pallas kernel here")


# speedup vs baseline: 1.9999x; 1.9999x over previous
import jax
import jax.numpy as jnp
from jax import lax
from jax.experimental import pallas as pl
from jax.experimental.pallas import tpu as pltpu
from jax.experimental.pallas import tpu_sc as plsc

D0, H, W = 21, 400, 352
D1 = (D0 - 3) // 2 + 1
D2 = (D1 - 3) // 2 + 1

NC, NSUB = 2, 16
NW = NC * NSUB
CH = 128
MB = 8192

SUBM_TAPS = tuple((dz, dy, dx) for dz in (-1, 0, 1) for dy in (-1, 0, 1)
                  for dx in (-1, 0, 1))
CONV_TAPS = ((0, 0, 0), (1, 0, 0), (2, 0, 0))

_INTERPRET = False


def _ceil_to(x, m):
    return (x + m - 1) // m * m


def _geom():
    hw = H * W
    cells0, cells1, cells2 = D0 * hw, D1 * hw, D2 * hw
    dp = _ceil_to(cells2, NW * CH)
    c0 = -(-(cells0 + 2) // (NSUB * MB))
    c1 = -(-(cells1 + 2) // (NSUB * MB))
    c2 = -(-(dp + 2) // (NSUB * MB))
    ga0, ga1, ga2 = NSUB * c0 * MB, NSUB * c1 * MB, NSUB * c2 * MB
    return hw, cells0, cells1, cells2, dp, (c0, c1, c2), (ga0, ga1, ga2)


def _mesh2():
    return plsc.VectorSubcoreMesh(core_axis_name="c", subcore_axis_name="s")


def _mesh1():
    return plsc.VectorSubcoreMesh(core_axis_name="c", subcore_axis_name="s",
                                  num_cores=1)


def _build_grids(n, NP0):
    hw, cells0, cells1, cells2, dp, (c0, c1, c2), (ga0, ga1, ga2) = _geom()
    NS1, NT2 = 2 * NP0, 4 * NP0
    S = n
    dump0, dump1, dump2 = ga0 - 8, ga1 - 8, ga2 - 8
    rpt = NP0 // NSUB
    nch = rpt // CH

    def body(zs, ys, xs, g0, g1, g2, zo1, zo2, mbuf, zv, yv, xv, ib, vb,
             zb1, zb2, sem):
        wid = lax.axis_index("s")

        @pl.loop(0, MB // 16)
        def _fill(i):
            mbuf[pl.ds(i * 16, 16)] = jnp.full((16,), S, jnp.int32)

        for grid, copies in ((g0, c0), (g1, c1), (g2, c2)):
            gbase = wid * copies * MB

            @pl.loop(0, copies)
            def _ms(j, grid=grid, gbase=gbase):
                pltpu.sync_copy(
                    mbuf, grid.at[pl.ds(pl.multiple_of(gbase + j * MB, MB),
                                        MB)])

        plsc.subcore_barrier()

        @pl.loop(0, nch)
        def _chunk(ci):
            base = pl.multiple_of(wid * rpt + ci * CH, CH)
            pltpu.sync_copy(zs.at[pl.ds(base, CH)], zv)
            pltpu.sync_copy(ys.at[pl.ds(base, CH)], yv)
            pltpu.sync_copy(xs.at[pl.ds(base, CH)], xv)
            for v in range(CH // 16):
                sl = pl.ds(v * 16, 16)
                z = zv[sl]
                yw = yv[sl] * W + xv[sl]
                lane = base + v * 16 + lax.iota(jnp.int32, 16)
                rv = z >= 0
                ib[0, sl] = jnp.where(rv, z * hw + yw, dump0)
                vb[0, sl] = lane
                zoa = z >> 1
                va = rv & (zoa < D1)
                zob = zoa - 1
                vbm = rv & ((z & 1) == 0) & (zob >= 0)
                zoa_m = jnp.where(va, zoa, -1)
                zob_m = jnp.where(vbm, zob, -1)
                zb1[0, sl] = zoa_m
                zb1[1, sl] = zob_m
                ib[1, sl] = jnp.where(va, zoa * hw + yw, dump1)
                vb[1, sl] = lane
                ib[2, sl] = jnp.where(vbm, zob * hw + yw, dump1)
                vb[2, sl] = NP0 + lane
                for w1, zo1v in ((0, zoa_m), (1, zob_m)):
                    z2a = zo1v >> 1
                    va2 = (zo1v >= 0) & (z2a < D2)
                    z2b = z2a - 1
                    vb2 = (zo1v >= 0) & ((zo1v & 1) == 0) & (z2b >= 0)
                    for w2, z2, vld in ((0, z2a, va2), (1, z2b, vb2)):
                        r = w2 * 2 + w1
                        off = w2 * NS1 + w1 * NP0
                        ib[3 + r, sl] = jnp.where(vld, z2 * hw + yw, dump2)
                        vb[3 + r, sl] = off + lane
                        zb2[r, sl] = jnp.where(vld, z2, -1)
            pltpu.sync_copy(zb1.at[0], zo1.at[pl.ds(base, CH)])
            pltpu.sync_copy(
                zb1.at[1], zo1.at[pl.ds(pl.multiple_of(NP0 + base, CH), CH)])
            for r, off in enumerate((0, NP0, NS1, NS1 + NP0)):
                pltpu.sync_copy(
                    zb2.at[r],
                    zo2.at[pl.ds(pl.multiple_of(off + base, CH), CH)])
            descs = []
            for j, grid in ((0, g0), (1, g1), (2, g1), (3, g2), (4, g2),
                            (5, g2), (6, g2)):
                descs.append(pltpu.async_copy(vb.at[j], grid.at[ib.at[j]],
                                              sem))
            for d in descs:
                d.wait()

    return pl.kernel(
        body,
        out_type=[
            jax.ShapeDtypeStruct((ga0,), jnp.int32),
            jax.ShapeDtypeStruct((ga1,), jnp.int32),
            jax.ShapeDtypeStruct((ga2,), jnp.int32),
            jax.ShapeDtypeStruct((NS1,), jnp.int32),
            jax.ShapeDtypeStruct((NT2,), jnp.int32),
        ],
        mesh=_mesh1(),
        scratch_types=[
            pltpu.VMEM((MB,), jnp.int32),
            pltpu.VMEM((CH,), jnp.int32),
            pltpu.VMEM((CH,), jnp.int32),
            pltpu.VMEM((CH,), jnp.int32),
            pltpu.VMEM((7, CH), jnp.int32),
            pltpu.VMEM((7, CH), jnp.int32),
            pltpu.VMEM((2, CH), jnp.int32),
            pltpu.VMEM((4, CH), jnp.int32),
            pltpu.SemaphoreType.DMA,
        ],
        compiler_params=pltpu.CompilerParams(use_tc_tiling_on_sc=False),
        interpret=_INTERPRET,
    )


def _make_idx(NR, NP0, GA, taps, zscale, d_in, sent_cell):
    hw = H * W
    K = len(taps)
    rpt = NR // NW
    nch = rpt // CH

    def body(grid, zarr, ys, xs, ridx, zv, yv, xv, ib, gv, sem):
        wid = lax.axis_index("s") * NC + lax.axis_index("c")

        @pl.loop(0, nch)
        def _chunk(ci):
            base = pl.multiple_of(wid * rpt + ci * CH, CH)
            hb = pl.multiple_of(base & (NP0 - 1), CH)
            pltpu.sync_copy(zarr.at[pl.ds(base, CH)], zv)
            pltpu.sync_copy(ys.at[pl.ds(hb, CH)], yv)
            pltpu.sync_copy(xs.at[pl.ds(hb, CH)], xv)
            for v in range(CH // 16):
                sl = pl.ds(v * 16, 16)
                z = zv[sl]
                y = yv[sl]
                x = xv[sl]
                rv = z >= 0
                zz = z * zscale
                fl0 = zz * hw + y * W + x
                mz = {dz: rv & (zz + dz >= 0) & (zz + dz < d_in)
                      for dz in set(t[0] for t in taps)}
                my = {dy: (y + dy >= 0) & (y + dy < H)
                      for dy in set(t[1] for t in taps)}
                mx = {dx: (x + dx >= 0) & (x + dx < W)
                      for dx in set(t[2] for t in taps)}
                for k, (dz, dy, dx) in enumerate(taps):
                    m = mz[dz] & my[dy] & mx[dx]
                    ib[k, sl] = jnp.where(m, fl0 + ((dz * H + dy) * W + dx),
                                          sent_cell)
            descs = [pltpu.async_copy(grid.at[ib.at[k]], gv.at[k], sem)
                     for k in range(K)]
            for d in descs:
                d.wait()
            pltpu.sync_copy(gv, ridx.at[wid * nch + ci])

    return pl.kernel(
        body,
        out_type=jax.ShapeDtypeStruct((NR // CH, K, CH), jnp.int32),
        mesh=_mesh2(),
        scratch_types=[
            pltpu.VMEM((CH,), jnp.int32),
            pltpu.VMEM((CH,), jnp.int32),
            pltpu.VMEM((CH,), jnp.int32),
            pltpu.VMEM((K, CH), jnp.int32),
            pltpu.VMEM((K, CH), jnp.int32),
            pltpu.SemaphoreType.DMA,
        ],
        compiler_params=pltpu.CompilerParams(use_tc_tiling_on_sc=False),
        interpret=_INTERPRET,
    )


def _make_rowgather(NR, C, K):
    rpt = NR // NW
    nch = rpt // CH

    def body(*refs):
        ridx, xsrc = refs[0], refs[1]
        gs = refs[2:2 + K]
        ib, rb0, rb1, sem0, sem1 = refs[2 + K:]
        wid = lax.axis_index("s") * NC + lax.axis_index("c")
        rbs = (rb0, rb1)
        sems = (sem0, sem1)

        @pl.loop(0, nch)
        def _chunk(ci):
            base = pl.multiple_of(wid * rpt + ci * CH, CH)
            pltpu.sync_copy(ridx.at[wid * nch + ci], ib)
            descs = {0: pltpu.async_copy(xsrc.at[ib.at[0]], rb0, sem0)}
            for k in range(K):
                if k + 1 < K:
                    descs[k + 1] = pltpu.async_copy(
                        xsrc.at[ib.at[k + 1]], rbs[(k + 1) % 2],
                        sems[(k + 1) % 2])
                descs[k].wait()
                pltpu.sync_copy(rbs[k % 2], gs[k].at[pl.ds(base, CH)])

    return pl.kernel(
        body,
        out_type=[jax.ShapeDtypeStruct((NR, C), jnp.float32)
                  for _ in range(K)],
        mesh=_mesh2(),
        scratch_types=[
            pltpu.VMEM((K, CH), jnp.int32),
            pltpu.VMEM((CH, C), jnp.float32),
            pltpu.VMEM((CH, C), jnp.float32),
            pltpu.SemaphoreType.DMA,
            pltpu.SemaphoreType.DMA,
        ],
        compiler_params=pltpu.CompilerParams(use_tc_tiling_on_sc=False),
        interpret=_INTERPRET,
    )


def _make_final(DP, GA2):
    rpt = DP // NW
    nch = rpt // CH

    def body(g2, x3, dense, ib, rb0, rb1, sem0, sem1):
        wid = lax.axis_index("s") * NC + lax.axis_index("c")
        rbs = (rb0, rb1)
        sems = (sem0, sem1)

        @pl.loop(0, nch)
        def _chunk(ci):
            base = pl.multiple_of(wid * rpt + ci * CH, CH)
            pltpu.sync_copy(g2.at[pl.ds(base, CH)], ib.at[0])
            pltpu.async_copy(x3.at[ib.at[0]], rb0, sem0).wait()
            pltpu.sync_copy(rb0, dense.at[pl.ds(base, CH)])

    return pl.kernel(
        body,
        out_type=jax.ShapeDtypeStruct((DP, 32), jnp.float32),
        mesh=_mesh2(),
        scratch_types=[
            pltpu.VMEM((1, CH), jnp.int32),
            pltpu.VMEM((CH, 32), jnp.float32),
            pltpu.VMEM((CH, 32), jnp.float32),
            pltpu.SemaphoreType.DMA,
            pltpu.SemaphoreType.DMA,
        ],
        compiler_params=pltpu.CompilerParams(use_tc_tiling_on_sc=False),
        interpret=_INTERPRET,
    )


def _mm_relu(gs, Wall):
    K = len(gs)
    NR, C = gs[0].shape
    KC = K * C
    CO = Wall.shape[1]
    BR = 512 if KC > 512 else 4096

    def body(*refs):
        g_refs = refs[:K]
        w_ref, o_ref = refs[K], refs[K + 1]
        x = jnp.concatenate([g[...] for g in g_refs], axis=1)
        o_ref[...] = jax.nn.relu(
            jnp.dot(x, w_ref[...], preferred_element_type=jnp.float32))

    return pl.pallas_call(
        body,
        grid=(NR // BR,),
        in_specs=[pl.BlockSpec((BR, C), lambda r: (r, 0))
                  for _ in range(K)] +
                 [pl.BlockSpec((KC, CO), lambda r: (0, 0))],
        out_specs=pl.BlockSpec((BR, CO), lambda r: (r, 0)),
        out_shape=jax.ShapeDtypeStruct((NR, CO), jnp.float32),
        interpret=_INTERPRET,
    )(*gs, Wall)


def _transpose_tc(dense3):
    d2, hw, co = dense3.shape
    BP = 1280
    assert hw % BP == 0

    def body(x_ref, o_ref):
        for z in range(d2):
            o_ref[:, z, :] = x_ref[z].T

    return pl.pallas_call(
        body,
        grid=(hw // BP,),
        in_specs=[pl.BlockSpec((d2, BP, co), lambda p: (0, p, 0))],
        out_specs=pl.BlockSpec((co, d2, BP), lambda p: (0, 0, p)),
        out_shape=jax.ShapeDtypeStruct((co, d2, hw), jnp.float32),
        interpret=_INTERPRET,
    )(dense3)


def kernel(voxel_features, coors, batch_size, W_subm0, W_conv1, W_subm1,
           W_subm2, W_conv2):
    n, C0 = voxel_features.shape
    hw, cells0, cells1, cells2, DP, _, (ga0, ga1, ga2) = _geom()
    NP0 = _ceil_to(n + 1, NW * CH)
    NS1, NT2 = 2 * NP0, 4 * NP0

    zpad = jnp.full((NP0 - n,), -1, jnp.int32)
    opad = jnp.zeros((NP0 - n,), jnp.int32)
    z0 = jnp.concatenate([coors[:, 1].astype(jnp.int32), zpad])
    y0 = jnp.concatenate([coors[:, 2].astype(jnp.int32), opad])
    x0 = jnp.concatenate([coors[:, 3].astype(jnp.int32), opad])
    feats = jnp.zeros((NP0, C0), jnp.float32).at[:n].set(voxel_features)

    g0, g1, g2, zo1, zo2 = _build_grids(n, NP0)(z0, y0, x0)

    ridx0 = _make_idx(NP0, NP0, ga0, SUBM_TAPS, 1, D0, cells0)(g0, z0, y0, x0)
    G0 = _make_rowgather(NP0, C0, 27)(ridx0, feats)
    x_0 = _mm_relu(G0, W_subm0.reshape(27 * C0, 32))

    rc1 = _make_idx(NS1, NP0, ga0, CONV_TAPS, 2, D0, cells0)(g0, zo1, y0, x0)
    Gc1 = _make_rowgather(NS1, 32, 3)(rc1, x_0)
    x_1 = _mm_relu(Gc1, W_conv1.reshape(96, 32))

    ridx1 = _make_idx(NS1, NP0, ga1, SUBM_TAPS, 1, D1, cells1)(g1, zo1, y0, x0)
    G1a = _make_rowgather(NS1, 32, 27)(ridx1, x_1)
    x_1a = _mm_relu(G1a, W_subm1.reshape(27 * 32, 32))
    G1b = _make_rowgather(NS1, 32, 27)(ridx1, x_1a)
    x_1b = _mm_relu(G1b, W_subm2.reshape(27 * 32, 32))

    rc2 = _make_idx(NT2, NP0, ga1, CONV_TAPS, 2, D1, cells1)(g1, zo2, y0, x0)
    Gc2 = _make_rowgather(NT2, 32, 3)(rc2, x_1b)
    x_2 = _mm_relu(Gc2, W_conv2.reshape(96, 32))

    dense = _make_final(DP, ga2)(g2, x_2)
    out3 = _transpose_tc(dense[:cells2].reshape(D2, hw, 32))
    return out3.reshape(1, 32 * D2, H, W)
